# Initial kernel scaffold; baseline (speedup 1.0000x reference)
import jax
import jax.numpy as jnp
from jax.experimental import pallas as pl


def kernel(eeg_dem_scores, mmse, W1, a_src1, a_dst1, b1, W2, a_src2, a_dst2, b2, Wm, bm, Wc, bc):
    raise NotImplementedError("write your pallas kernel here")


# speedup vs baseline: 5.2760x; 5.2760x over previous
import functools

import jax
import jax.numpy as jnp
from jax import lax
from jax.experimental import pallas as pl
from jax.experimental.pallas import tpu as pltpu
from jax.experimental.pallas import tpu_sc as plsc

N = 19
L = 16
NEG = -1e30

OFF_X = 0
OFF_W1 = 32
OFF_AS1 = 96
OFF_AD1 = 160
OFF_B1 = 224
OFF_W2 = 288
OFF_AS2 = 8480
OFF_AD2 = 8608
OFF_B2 = 8736
OFF_WCR = 8864
OFF_WCM = 11296
OFF_WM = 11328
OFF_BM = 11360
OFF_SCAL = 11392
TOT = 11408

S_AS2 = 0
S_ALPHA = 64


def _lrelu(t):
    return jnp.where(t >= 0.0, t, 0.2 * t)


def _body(packed_hbm, out_hbm, buf, h_ref, m_ref, s_ref, out_v):
    cid = lax.axis_index("c")
    sid = lax.axis_index("s")

    @pl.when(jnp.logical_and(cid == 0, sid == 0))
    def _():
        pltpu.sync_copy(packed_hbm, buf)

        def vl(off):
            return buf[pl.ds(off, L)]

        xa = vl(OFF_X)
        xb = vl(OFF_X + L)

        acc_s = vl(OFF_W1) * vl(OFF_AS1)
        acc_d = vl(OFF_W1) * vl(OFF_AD1)
        for k in range(1, 4):
            acc_s = acc_s + vl(OFF_W1 + 16 * k) * vl(OFF_AS1 + 16 * k)
            acc_d = acc_d + vl(OFF_W1 + 16 * k) * vl(OFF_AD1 + 16 * k)
        cs1 = jnp.sum(acc_s)
        cd1 = jnp.sum(acc_d)

        as1a = xa * cs1
        as1b = xb * cs1
        lane = lax.iota(jnp.int32, L)
        tail_mask = lane < (N - L)

        def bcast(ref, idx):
            return plsc.load_gather(ref, [jnp.full((L,), idx, jnp.int32)])

        lane0 = lane == 0

        ys = []
        for i in range(N):
            adi = (xa[i] if i < L else xb[i - L]) * cd1
            e_a = _lrelu(as1a + adi)
            e_b = jnp.where(tail_mask, _lrelu(as1b + adi), NEG)
            m = jnp.maximum(jnp.max(e_a), jnp.max(e_b))
            p_a = jnp.exp(e_a - m)
            p_b = jnp.exp(e_b - m)
            s = jnp.sum(p_a) + jnp.sum(p_b)
            num = jnp.sum(p_a * xa) + jnp.sum(p_b * xb)
            ys.append(jnp.broadcast_to(num, (L,)) / jnp.broadcast_to(s, (L,)))

        w1k = [vl(OFF_W1 + 16 * k) for k in range(4)]
        b1k = [vl(OFF_B1 + 16 * k) for k in range(4)]
        for i in range(N):
            for k in range(4):
                h_ref[pl.ds(i * 64 + 16 * k, L)] = jnp.maximum(
                    ys[i] * w1k[k] + b1k[k], 0.0)

        zero = jnp.zeros((L,), jnp.float32)

        def vsvd_step(c, carry):
            base = OFF_W2 + c * 64
            ss = bcast(buf, OFF_AS2 + c)
            sd = bcast(buf, OFF_AD2 + c)
            out = []
            for k in range(4):
                w = buf[pl.ds(base + 16 * k, L)]
                out.append(carry[k] + ss * w)
                out.append(carry[4 + k] + sd * w)
            return tuple(out[0::2]) + tuple(out[1::2])

        vsvd = lax.fori_loop(0, 128, vsvd_step, (zero,) * 8)
        vs2 = vsvd[:4]
        vd2 = vsvd[4:]

        s_ref[pl.ds(S_AS2 + 16, L)] = jnp.full((L,), NEG, jnp.float32)
        ad2 = []
        for i in range(N):
            hk = [h_ref[pl.ds(i * 64 + 16 * k, L)] for k in range(4)]
            a_s = hk[0] * vs2[0]
            a_d = hk[0] * vd2[0]
            for k in range(1, 4):
                a_s = a_s + hk[k] * vs2[k]
                a_d = a_d + hk[k] * vd2[k]
            plsc.store_scatter(
                s_ref, [jnp.full((L,), S_AS2 + i, jnp.int32)],
                jnp.broadcast_to(jnp.sum(a_s), (L,)), mask=lane0)
            ad2.append(jnp.sum(a_d))

        as2a = s_ref[pl.ds(S_AS2, L)]
        as2b = s_ref[pl.ds(S_AS2 + L, L)]

        for i in range(N):
            e_a = _lrelu(as2a + ad2[i])
            e_b = _lrelu(as2b + ad2[i])
            m = jnp.maximum(jnp.max(e_a), jnp.max(e_b))
            p_a = jnp.exp(e_a - m)
            p_b = jnp.exp(e_b - m)
            rs = 1.0 / jnp.broadcast_to(jnp.sum(p_a) + jnp.sum(p_b), (L,))
            s_ref[pl.ds(S_ALPHA, L)] = p_a
            s_ref[pl.ds(S_ALPHA + L, L)] = p_b

            def m_step(j, carry):
                pj = bcast(s_ref, S_ALPHA + j)
                return tuple(
                    carry[k] + pj * h_ref[pl.ds(j * 64 + 16 * k, L)]
                    for k in range(4))

            mk = lax.fori_loop(0, N, m_step, (zero,) * 4)
            for k in range(4):
                m_ref[pl.ds(i * 64 + 16 * k, L)] = mk[k] * rs

        p1v = zero
        for rows in (range(0, 8), range(8, 16), range(16, 19)):
            rows = list(rows)
            G = len(rows)

            def u_step(c, carry, rows=rows, G=G):
                base = OFF_W2 + c * 64
                wk = [buf[pl.ds(base + 16 * k, L)] for k in range(4)]
                out = list(carry)
                for r, i in enumerate(rows):
                    s = bcast(buf, OFF_WCR + i * 128 + c)
                    for k in range(4):
                        out[r * 4 + k] = out[r * 4 + k] + s * wk[k]
                return tuple(out)

            uacc = lax.fori_loop(0, 128, u_step, (zero,) * (4 * G))
            for r, i in enumerate(rows):
                for k in range(4):
                    p1v = p1v + uacc[r * 4 + k] * m_ref[pl.ds(i * 64 + 16 * k, L)]

        def col_step(i, carry):
            return tuple(
                carry[k] + buf[pl.ds(OFF_WCR + i * 128 + 16 * k, L)]
                for k in range(8))

        cols = lax.fori_loop(0, N, col_step, (zero,) * 8)
        for k in range(8):
            p1v = p1v + cols[k] * vl(OFF_B2 + 16 * k)

        scal = vl(OFF_SCAL)
        mmse = scal[0]
        for k in range(2):
            t = mmse * vl(OFF_WM + 16 * k) + vl(OFF_BM + 16 * k)
            p1v = p1v + t * vl(OFF_WCM + 16 * k)

        z = jnp.sum(p1v) + scal[1]
        zv = jnp.broadcast_to(z, (L,))
        out_v[...] = 1.0 / (1.0 + jnp.exp(-zv))
        pltpu.sync_copy(out_v, out_hbm)


@jax.jit
def _run(packed):
    mesh = plsc.VectorSubcoreMesh(
        core_axis_name="c", subcore_axis_name="s", num_cores=2,
        num_subcores=16)
    f = pl.kernel(
        _body,
        out_type=jax.ShapeDtypeStruct((L,), jnp.float32),
        mesh=mesh,
        compiler_params=pltpu.CompilerParams(needs_layout_passes=False),
        scratch_types=[
            pltpu.VMEM((TOT,), jnp.float32),
            pltpu.VMEM((N * 64,), jnp.float32),
            pltpu.VMEM((N * 64,), jnp.float32),
            pltpu.VMEM((128,), jnp.float32),
            pltpu.VMEM((L,), jnp.float32),
        ],
    )
    return f(packed)


def kernel(eeg_dem_scores, mmse, W1, a_src1, a_dst1, b1, W2, a_src2,
           a_dst2, b2, Wm, bm, Wc, bc):
    x = eeg_dem_scores[:, 0].astype(jnp.float32)
    scal = jnp.zeros((L,), jnp.float32).at[0].set(mmse[0]).at[1].set(bc[0])
    packed = jnp.concatenate([
        x, jnp.zeros((32 - N,), jnp.float32),
        W1[:, 0], a_src1, a_dst1, b1,
        W2.reshape(-1),
        a_src2, a_dst2, b2,
        Wc[0, :N * 128], Wc[0, N * 128:],
        Wm[:, 0], bm, scal,
    ])
    out = _run(packed)
    return out[0:1].reshape(1, 1)
